# baseline (speedup 1.0000x reference)
import jax
import jax.numpy as jnp
from jax import lax
from jax.experimental import pallas as pl
from jax.experimental.pallas import tpu as pltpu
from jax.experimental.pallas import tpu_sc as plsc

_R = 7
_K = 2 * _R + 1
_H = 64
_W = 64
_HW = _H * _W
_NC = 2
_NS = 16
_L = 16
_SLOTS = 96
_SW = _SLOTS * _W
_IPC = _H // _NC
_CBT = _HW // 128 // _NS
_BW = 16 * _W


def _sc_body(z_hbm, b_hbm, out_hbm, bvm, band8, s0_sh, s64_sh, sem):
    c = lax.axis_index("c")
    s = lax.axis_index("s")
    ph = s // 8
    g = s % 8
    rg = pl.multiple_of(g * 8, 8)
    pltpu.sync_copy(b_hbm, bvm)

    slot00 = 56 - (1 - c) * 32
    slot0 = slot00 - ph
    ws = pl.multiple_of((slot0 - (slot0 & 1)) * _W, 128)

    @pl.when(ph == 0)
    def _():
        pltpu.sync_copy(z_hbm, s0_sh.at[pl.ds(rg, 8), :])

    @pl.when(ph == 1)
    def _():
        pltpu.sync_copy(z_hbm, s64_sh.at[pl.ds(rg, 8), :])

    plsc.subcore_barrier()

    lo0 = slot00 * _W
    lo64 = lo0 - _W

    def fire(ii, band_pass):
        i_ev = c * _IPC + 2 * ii
        base = (15 - ii) * 128
        for k in range(_CBT):
            cb = s * _CBT + k
            w0 = base + cb * 128
            srcc = pl.multiple_of(w0, 128)
            dst_col = pl.multiple_of(cb * 128, 128)
            hit64 = jnp.logical_and(w0 < lo64 + 15 * _W, w0 + 128 > lo64)
            hit0 = jnp.logical_and(w0 < lo0 + 15 * _W, w0 + 128 > lo0)

            @pl.when(hit64 == band_pass)
            def _():
                pltpu.async_copy(
                    s64_sh.at[:, pl.ds(srcc, 128)],
                    out_hbm.at[
                        pl.ds(pl.multiple_of(i_ev * _W, 64), _H),
                        pl.ds(dst_col, 128),
                    ],
                    sem,
                )

            @pl.when(hit0 == band_pass)
            def _():
                pltpu.async_copy(
                    s0_sh.at[:, pl.ds(srcc, 128)],
                    out_hbm.at[
                        pl.ds(pl.multiple_of((i_ev + 1) * _W, 64), _H),
                        pl.ds(dst_col, 128),
                    ],
                    sem,
                )

    def early_body(ii, carry):
        fire(ii, False)
        return carry

    lax.fori_loop(0, _IPC // 2, early_body, 0)

    zeros16 = jnp.zeros((_L,), jnp.float32)

    def zero_body(t, carry):
        for u in range(8):
            band8[t // 8, pl.ds((t % 8) * 128 + u * _L, _L)] = zeros16
        return carry

    lax.fori_loop(0, 8 * _BW // 128, zero_body, 0)

    lane = lax.iota(jnp.int32, _L)
    off0 = (slot0 & 1) * _W

    def fill_body(t, carry):
        jl = t // _K
        a = t % _K
        j = g * 8 + jl
        row = bvm[pl.ds(a * _L, _L)]
        for m in range(_W // _L):
            q = m * _L + lane
            b = q - j + _R
            inb = jnp.logical_and(b >= 0, b < _K)
            bcl = jnp.clip(b, 0, _K - 1)
            vals = jnp.where(inb, row.at[bcl].get(mode="promise_in_bounds"), 0.0)
            band8[jl, pl.ds(off0 + a * _W + m * _L, _L)] = vals
        return carry

    lax.fori_loop(0, 8 * _K, fill_body, 0)

    @pl.when(ph == 0)
    def _():
        pltpu.sync_copy(band8, s0_sh.at[pl.ds(rg, 8), pl.ds(ws, _BW)])

    @pl.when(ph == 1)
    def _():
        pltpu.sync_copy(band8, s64_sh.at[pl.ds(rg, 8), pl.ds(ws, _BW)])

    plsc.subcore_barrier()

    def late_body(ii, carry):
        fire(ii, True)
        return carry

    lax.fori_loop(0, _IPC // 2, late_body, 0)

    def drain_body(k, carry):
        pltpu.make_async_copy(
            out_hbm.at[pl.ds(0, _H), pl.ds(0, 128)],
            s0_sh.at[:, pl.ds(0, 128)],
            sem,
        ).wait()
        return carry

    lax.fori_loop(0, _IPC * _CBT, drain_body, 0)


def kernel(feat_shape, biases):
    del feat_shape
    mesh = plsc.VectorSubcoreMesh(
        core_axis_name="c", subcore_axis_name="s",
        num_cores=_NC, num_subcores=_NS,
    )
    run = pl.kernel(
        _sc_body,
        out_type=jax.ShapeDtypeStruct((_HW, _HW), jnp.float32),
        mesh=mesh,
        scratch_types=[
            pltpu.VMEM((_K * _L,), jnp.float32),
            pltpu.VMEM((8, _BW), jnp.float32),
            pltpu.VMEM_SHARED((_H, _SW), jnp.float32),
            pltpu.VMEM_SHARED((_H, _SW), jnp.float32),
            pltpu.SemaphoreType.DMA,
        ],
    )
    zeros = jnp.zeros((8, _SW), jnp.float32)
    b_pad = jnp.pad(biases, ((0, 0), (0, _L - _K))).reshape(_K * _L)
    return run(zeros, b_pad)
